# baseline (speedup 1.0000x reference)
import functools
import math

import jax
import jax.numpy as jnp
from jax import lax
from jax.experimental import pallas as pl
from jax.experimental.pallas import tpu as pltpu
from jax.experimental.pallas import tpu_sc as plsc

_NC = 2
_NS = 16
_NW = _NC * _NS


def _sc_gather_fn(N, D, KP, rpw, chunk):
    nchunks = rpw // chunk
    mesh = plsc.VectorSubcoreMesh(core_axis_name="c", subcore_axis_name="s")

    @functools.partial(
        pl.kernel,
        out_type=(
            jax.ShapeDtypeStruct((N, D), jnp.float32),
            jax.ShapeDtypeStruct((N,), jnp.float32),
            jax.ShapeDtypeStruct((KP, D), jnp.float32),
            jax.ShapeDtypeStruct((KP,), jnp.float32),
        ),
        mesh=mesh,
        scratch_types=[
            pltpu.VMEM((rpw,), jnp.int32),
            pltpu.VMEM((chunk, D), jnp.float32),
            pltpu.VMEM((rpw,), jnp.float32),
            pltpu.VMEM((KP,), jnp.int32),
            pltpu.VMEM((KP,), jnp.float32),
            pltpu.SemaphoreType.DMA,
        ],
    )
    def sc_gather(w_hbm, tgt_hbm, bias_hbm, nids_hbm,
                  emb_out, bt_out, ne_out, nb_out,
                  idx_v, rows_v, biasg_v, nidx_v, nbias_v, sem):
        wid = lax.axis_index("s") * _NC + lax.axis_index("c")
        base = wid * rpw
        pltpu.sync_copy(tgt_hbm.at[pl.ds(base, rpw)], idx_v)

        @pl.when(wid == 0)
        def _():
            pltpu.sync_copy(nids_hbm, nidx_v)
            pltpu.async_copy(w_hbm.at[nidx_v], rows_v.at[pl.ds(0, KP)], sem).wait()
            pltpu.sync_copy(rows_v.at[pl.ds(0, KP)], ne_out)
            pltpu.async_copy(bias_hbm.at[nidx_v], nbias_v, sem).wait()
            pltpu.sync_copy(nbias_v, nb_out)

        for ci in range(nchunks):
            pltpu.async_copy(
                w_hbm.at[idx_v.at[pl.ds(ci * chunk, chunk)]], rows_v, sem
            ).wait()
            pltpu.sync_copy(rows_v, emb_out.at[pl.ds(base + ci * chunk, chunk)])

        for ci in range(rpw // 128):
            pltpu.async_copy(
                bias_hbm.at[idx_v.at[pl.ds(ci * 128, 128)]],
                biasg_v.at[pl.ds(ci * 128, 128)], sem,
            ).wait()
        pltpu.sync_copy(biasg_v, bt_out.at[pl.ds(base, rpw)])

    return sc_gather


def _softplus(x):
    return jnp.maximum(x, 0.0) + jnp.log1p(jnp.exp(-jnp.abs(x)))


def _tc_loss_fn(N, D, K, KP, R, const):
    nblk = N // R

    def body(h_ref, e_ref, bt_ref, ne_ref, nb_ref, out_ref):
        i = pl.program_id(0)
        h = h_ref[...]
        e = e_ref[...]
        s_true = jnp.sum(h * e, axis=1) + bt_ref[...].reshape(R)
        loss_true = _softplus(-(s_true + const))
        s_noise = lax.dot_general(
            h, ne_ref[...], (((1,), (1,)), ((), ())),
            preferred_element_type=jnp.float32,
        ) + nb_ref[...].reshape(1, KP)
        loss_noise = _softplus(s_noise + const)
        mask = lax.broadcasted_iota(jnp.int32, (R, KP), 1) < K
        loss_noise = jnp.where(mask, loss_noise, 0.0)
        total = jnp.sum(loss_true) + jnp.sum(loss_noise)

        @pl.when(i == 0)
        def _():
            out_ref[...] = jnp.zeros_like(out_ref)

        out_ref[...] += total.reshape(1, 1)

    grid_spec = pl.GridSpec(
        grid=(nblk,),
        in_specs=[
            pl.BlockSpec((R, D), lambda i: (i, 0)),
            pl.BlockSpec((R, D), lambda i: (i, 0)),
            pl.BlockSpec((1, R // 128, 128), lambda i: (i, 0, 0)),
            pl.BlockSpec((KP, D), lambda i: (0, 0)),
            pl.BlockSpec((1, KP), lambda i: (0, 0)),
        ],
        out_specs=pl.BlockSpec((1, 1), lambda i: (0, 0)),
    )
    return pl.pallas_call(
        body,
        grid_spec=grid_spec,
        out_shape=jax.ShapeDtypeStruct((1, 1), jnp.float32),
    )


def kernel(hidden, targets, W_emb, bias, noise_ids):
    B, S, D = hidden.shape
    N = B * S
    K = noise_ids.shape[0]
    KP = 32
    V = W_emb.shape[0]

    tgt = targets.reshape(N).astype(jnp.int32)
    nids = jnp.pad(noise_ids.astype(jnp.int32), (0, KP - K))
    hid2 = hidden.reshape(N, D)

    rpw = N // _NW
    chunk = 64
    emb_true, bias_true, noise_emb, noise_bias = _sc_gather_fn(N, D, KP, rpw, chunk)(
        W_emb, tgt, bias, nids
    )

    const = math.log(float(V)) - math.log(float(K))
    R = 512
    total = _tc_loss_fn(N, D, K, KP, R, const)(
        hid2, emb_true, bias_true.reshape(N // R, R // 128, 128),
        noise_emb, noise_bias.reshape(1, KP),
    )
    return total[0, 0] / N
